# baseline (speedup 1.0000x reference)
import functools

import jax
import jax.numpy as jnp
from jax import lax
from jax.experimental import pallas as pl
from jax.experimental.pallas import tpu as pltpu
from jax.experimental.pallas import tpu_sc as plsc

DIMS = 10000
LEVELS = 1000
POS = 784
BATCH = 8
CLASSES = 10

W = 80
NCHUNK = DIMS // W
NTILES = 32
PSTRIP = 112
NSTRIP = POS // PSTRIP
NV = W // 16


def _sc_encode_body(lev_hbm, pos_hbm, idx_hbm, out_hbm, ltab, ptab, idx_v,
                    accb):
    wid = lax.axis_index("s") * 2 + lax.axis_index("c")
    nchunks = jnp.where(wid < NCHUNK - 3 * NTILES, 4, 3)

    def chunk_body(ci, carry0):
        c = wid + NTILES * ci
        pltpu.sync_copy(lev_hbm.at[:, pl.ds(c * W, W)], ltab)
        zero = jnp.zeros((16,), jnp.float32)
        for b in range(BATCH):
            for j in range(NV):
                accb[b, pl.ds(j * 16, 16)] = zero

        def strip_body(s, carry1):
            pltpu.sync_copy(
                pos_hbm.at[pl.ds(s * PSTRIP, PSTRIP), pl.ds(c * W, W)], ptab)
            pltpu.sync_copy(idx_hbm.at[:, pl.ds(s * PSTRIP, PSTRIP)], idx_v)

            def pbody(pb, carry2):
                p0 = pb * 16
                iv = [idx_v[b, pl.ds(p0, 16)] for b in range(BATCH)]
                for k in range(16):
                    pv = [ptab[p0 + k, pl.ds(j * 16, 16)] for j in range(NV)]
                    for b in range(BATCH):
                        i = iv[b][k]
                        for j in range(NV):
                            lv = ltab[i, pl.ds(j * 16, 16)]
                            plsc.addupdate(accb.at[b, pl.ds(j * 16, 16)],
                                           lv * pv[j])
                return carry2

            lax.fori_loop(0, PSTRIP // 16, pbody, 0)
            return carry1

        lax.fori_loop(0, NSTRIP, strip_body, 0)
        pltpu.sync_copy(accb, out_hbm.at[:, pl.ds(c * W, W)])
        return carry0

    lax.fori_loop(0, nchunks, chunk_body, 0)


def _finish_body(hv_ref, cw_ref, out_ref):
    enc = jnp.where(hv_ref[...] > 0, 1.0, -1.0).astype(jnp.float32)
    out_ref[...] = jax.lax.dot_general(
        enc, cw_ref[...], (((1,), (1,)), ((), ())),
        preferred_element_type=jnp.float32)


def _sc_encode(level_weight, position_weight, idx):
    mesh = plsc.VectorSubcoreMesh(core_axis_name="c", subcore_axis_name="s")
    sc_encode = pl.kernel(
        _sc_encode_body,
        out_type=jax.ShapeDtypeStruct((BATCH, DIMS), jnp.float32),
        mesh=mesh,
        compiler_params=pltpu.CompilerParams(use_tc_tiling_on_sc=False),
        scratch_types=[
            pltpu.VMEM((LEVELS, W), jnp.float32),
            pltpu.VMEM((PSTRIP, W), jnp.float32),
            pltpu.VMEM((BATCH, PSTRIP), jnp.int32),
            pltpu.VMEM((BATCH, W), jnp.float32),
        ],
    )
    return sc_encode(level_weight, position_weight, idx)


sc_encode_for_test = jax.jit(_sc_encode)


@jax.jit
def kernel(x, position_weight, level_weight, classify_weight):
    xf = x.reshape(BATCH, POS)
    idx = jnp.clip(jnp.round(xf * (LEVELS - 1)), 0, LEVELS - 1).astype(jnp.int32)
    sample_hv = _sc_encode(level_weight, position_weight, idx)

    logit = pl.pallas_call(
        _finish_body,
        out_shape=jax.ShapeDtypeStruct((BATCH, CLASSES), jnp.float32),
    )(sample_hv, classify_weight)
    return logit


# speedup vs baseline: 1.5638x; 1.5638x over previous
import functools

import jax
import jax.numpy as jnp
from jax import lax
from jax.experimental import pallas as pl
from jax.experimental.pallas import tpu as pltpu
from jax.experimental.pallas import tpu_sc as plsc

DIMS = 10000
LEVELS = 1000
POS = 784
BATCH = 8
CLASSES = 10

W = 80
NCHUNK = DIMS // W
NTILES = 32
PSTRIP = 56
NSTRIP = POS // PSTRIP
NV = W // 16
NIV = POS // 16


def _sc_encode_body(lev_hbm, pos_hbm, idx_hbm, out_hbm,
                    idxb, ilist, lrA, lrB, ptA, ptB, accb, semA, semB):
    wid = lax.axis_index("s") * 2 + lax.axis_index("c")
    nchunks = jnp.where(wid < NCHUNK - 3 * NTILES, 4, 3)
    pltpu.sync_copy(idx_hbm, idxb)

    def chunk_body(ci, carry0):
        c = wid + NTILES * ci

        for b in range(BATCH):
            for v in range(NIV):
                sl = pl.ds(v * 16, 16)
                ilist[b, sl] = idxb[b, sl] * NCHUNK + c

        zero = jnp.zeros((16,), jnp.float32)
        for b in range(BATCH):
            for j in range(NV):
                accb[b, pl.ds(j * 16, 16)] = zero

        def copies(s, lr, pt, sem):
            cps = [
                pltpu.make_async_copy(
                    lev_hbm.at[ilist.at[b, pl.ds(s * PSTRIP, PSTRIP)]],
                    lr.at[b], sem)
                for b in range(BATCH)
            ]
            cps.append(pltpu.make_async_copy(
                pos_hbm.at[pl.ds(s * PSTRIP, PSTRIP), pl.ds(c * W, W)],
                pt, sem))
            return cps

        def issue(s, lr, pt, sem):
            for cp in copies(s, lr, pt, sem):
                cp.start()

        def drain(s, lr, pt, sem):
            for cp in copies(s, lr, pt, sem):
                cp.wait()

        def compute(lr, pt):
            def pbody(p, carry2):
                pv = [pt[p, pl.ds(j * 16, 16)] for j in range(NV)]
                for b in range(BATCH):
                    for j in range(NV):
                        lv = lr[b, p, pl.ds(j * 16, 16)]
                        plsc.addupdate(accb.at[b, pl.ds(j * 16, 16)],
                                       lv * pv[j])
                return carry2

            lax.fori_loop(0, PSTRIP, pbody, 0)

        issue(0, lrA, ptA, semA)

        def ring(g, carry1):
            s0 = 2 * g
            issue(s0 + 1, lrB, ptB, semB)
            drain(s0, lrA, ptA, semA)
            compute(lrA, ptA)

            @pl.when(s0 + 2 < NSTRIP)
            def _():
                issue(s0 + 2, lrA, ptA, semA)

            drain(s0 + 1, lrB, ptB, semB)
            compute(lrB, ptB)
            return carry1

        lax.fori_loop(0, NSTRIP // 2, ring, 0)
        pltpu.sync_copy(accb, out_hbm.at[:, pl.ds(c * W, W)])
        return carry0

    lax.fori_loop(0, nchunks, chunk_body, 0)


def _finish_body(hv_ref, cw_ref, out_ref):
    enc = jnp.where(hv_ref[...] > 0, 1.0, -1.0).astype(jnp.float32)
    out_ref[...] = jax.lax.dot_general(
        enc, cw_ref[...], (((1,), (1,)), ((), ())),
        preferred_element_type=jnp.float32)


def _sc_encode(level_weight, position_weight, idx):
    lev_rows = level_weight.reshape(LEVELS * NCHUNK, W)
    mesh = plsc.VectorSubcoreMesh(core_axis_name="c", subcore_axis_name="s")
    sc_encode = pl.kernel(
        _sc_encode_body,
        out_type=jax.ShapeDtypeStruct((BATCH, DIMS), jnp.float32),
        mesh=mesh,
        compiler_params=pltpu.CompilerParams(use_tc_tiling_on_sc=False),
        scratch_types=[
            pltpu.VMEM((BATCH, POS), jnp.int32),
            pltpu.VMEM((BATCH, POS), jnp.int32),
            pltpu.VMEM((BATCH, PSTRIP, W), jnp.float32),
            pltpu.VMEM((BATCH, PSTRIP, W), jnp.float32),
            pltpu.VMEM((PSTRIP, W), jnp.float32),
            pltpu.VMEM((PSTRIP, W), jnp.float32),
            pltpu.VMEM((BATCH, W), jnp.float32),
            pltpu.SemaphoreType.DMA,
            pltpu.SemaphoreType.DMA,
        ],
    )
    return sc_encode(lev_rows, position_weight, idx)


sc_encode_for_test = jax.jit(_sc_encode)


@jax.jit
def kernel(x, position_weight, level_weight, classify_weight):
    xf = x.reshape(BATCH, POS)
    idx = jnp.clip(jnp.round(xf * (LEVELS - 1)), 0, LEVELS - 1).astype(jnp.int32)
    sample_hv = _sc_encode(level_weight, position_weight, idx)

    logit = pl.pallas_call(
        _finish_body,
        out_shape=jax.ShapeDtypeStruct((BATCH, CLASSES), jnp.float32),
    )(sample_hv, classify_weight)
    return logit


# speedup vs baseline: 4.3973x; 2.8119x over previous
import functools

import jax
import jax.numpy as jnp
from jax import lax
from jax.experimental import pallas as pl
from jax.experimental.pallas import tpu as pltpu
from jax.experimental.pallas import tpu_sc as plsc

DIMS = 10000
LEVELS = 1000
POS = 784
BATCH = 8
CLASSES = 10

W = 80
NCHUNK = DIMS // W
NTILES = 32
PSTRIP = 56
NSTRIP = POS // PSTRIP
NV = W // 16

DTC = 8320
DBLK = 640
C0 = DTC // W
NSC = NCHUNK - C0


def _sc_encode_body(lev_hbm, pos_hbm, idx_hbm, out_hbm,
                    idxb, ilist, lrA, lrB, ptA, ptB, accb, semA, semB):
    wid = lax.axis_index("s") * 2 + lax.axis_index("c")
    nfull = NSC // NTILES
    nrem = NSC % NTILES
    nchunks = nfull + jnp.where(wid < nrem, 1, 0)
    pltpu.sync_copy(idx_hbm, idxb)

    def chunk_body(ci, carry0):
        c = C0 + wid + NTILES * ci

        for b in range(BATCH):
            for v in range(POS // 16):
                sl = pl.ds(v * 16, 16)
                ilist[b, sl] = idxb[b, sl] * NCHUNK + c

        zero = jnp.zeros((16,), jnp.float32)
        for b in range(BATCH):
            for j in range(NV):
                accb[b, pl.ds(j * 16, 16)] = zero

        def copies(s, lr, pt, sem):
            cps = [
                pltpu.make_async_copy(
                    lev_hbm.at[ilist.at[b, pl.ds(s * PSTRIP, PSTRIP)]],
                    lr.at[b], sem)
                for b in range(BATCH)
            ]
            cps.append(pltpu.make_async_copy(
                pos_hbm.at[pl.ds(s * PSTRIP, PSTRIP), pl.ds(c * W, W)],
                pt, sem))
            return cps

        def issue(s, lr, pt, sem):
            for cp in copies(s, lr, pt, sem):
                cp.start()

        def drain(s, lr, pt, sem):
            for cp in copies(s, lr, pt, sem):
                cp.wait()

        def compute(lr, pt):
            def pbody(p, carry2):
                pv = [pt[p, pl.ds(j * 16, 16)] for j in range(NV)]
                for b in range(BATCH):
                    for j in range(NV):
                        lv = lr[b, p, pl.ds(j * 16, 16)]
                        plsc.addupdate(accb.at[b, pl.ds(j * 16, 16)],
                                       lv * pv[j])
                return carry2

            lax.fori_loop(0, PSTRIP, pbody, 0)

        issue(0, lrA, ptA, semA)

        def ring(g, carry1):
            s0 = 2 * g
            issue(s0 + 1, lrB, ptB, semB)
            drain(s0, lrA, ptA, semA)
            compute(lrA, ptA)

            @pl.when(s0 + 2 < NSTRIP)
            def _():
                issue(s0 + 2, lrA, ptA, semA)

            drain(s0 + 1, lrB, ptB, semB)
            compute(lrB, ptB)
            return carry1

        lax.fori_loop(0, NSTRIP // 2, ring, 0)
        pltpu.sync_copy(accb, out_hbm.at[:, pl.ds((c - C0) * W, W)])
        return carry0

    lax.fori_loop(0, nchunks, chunk_body, 0)


def _sc_encode(level_weight, position_weight, idx):
    lev_rows = level_weight.reshape(LEVELS * NCHUNK, W)
    mesh = plsc.VectorSubcoreMesh(core_axis_name="c", subcore_axis_name="s")
    sc_encode = pl.kernel(
        _sc_encode_body,
        out_type=jax.ShapeDtypeStruct((BATCH, DIMS - DTC), jnp.float32),
        mesh=mesh,
        compiler_params=pltpu.CompilerParams(use_tc_tiling_on_sc=False),
        scratch_types=[
            pltpu.VMEM((BATCH, POS), jnp.int32),
            pltpu.VMEM((BATCH, POS), jnp.int32),
            pltpu.VMEM((BATCH, PSTRIP, W), jnp.float32),
            pltpu.VMEM((BATCH, PSTRIP, W), jnp.float32),
            pltpu.VMEM((PSTRIP, W), jnp.float32),
            pltpu.VMEM((PSTRIP, W), jnp.float32),
            pltpu.VMEM((BATCH, W), jnp.float32),
            pltpu.SemaphoreType.DMA,
            pltpu.SemaphoreType.DMA,
        ],
    )
    return sc_encode(lev_rows, position_weight, idx)


sc_encode_for_test = jax.jit(_sc_encode)


def _tc_encode_body(xt_ref, lev_ref, pos_ref, out_ref):
    lev_bf = lev_ref[...].astype(jnp.bfloat16)
    pos_blk = pos_ref[...]
    lvl_iota = jax.lax.broadcasted_iota(jnp.int32, (POS, LEVELS), 1)
    for b in range(BATCH):
        xb = xt_ref[:, b : b + 1]
        idx = jnp.clip(jnp.round(xb * (LEVELS - 1)), 0, LEVELS - 1).astype(jnp.int32)
        onehot = (idx == lvl_iota).astype(jnp.bfloat16)
        g = jnp.dot(onehot, lev_bf, preferred_element_type=jnp.float32)
        out_ref[b, :] = jnp.sum(g * pos_blk, axis=0)


def _finish_body(hv_tc_ref, hv_sc_ref, cw_ref, out_ref):
    cw = cw_ref[...]
    enc_tc = jnp.where(hv_tc_ref[...] > 0, 1.0, -1.0).astype(jnp.float32)
    enc_sc = jnp.where(hv_sc_ref[...] > 0, 1.0, -1.0).astype(jnp.float32)
    out_ref[...] = (
        jax.lax.dot_general(enc_tc, cw[:, :DTC], (((1,), (1,)), ((), ())),
                            preferred_element_type=jnp.float32)
        + jax.lax.dot_general(enc_sc, cw[:, DTC:], (((1,), (1,)), ((), ())),
                              preferred_element_type=jnp.float32))


@jax.jit
def kernel(x, position_weight, level_weight, classify_weight):
    xf = x.reshape(BATCH, POS)
    idx = jnp.clip(jnp.round(xf * (LEVELS - 1)), 0, LEVELS - 1).astype(jnp.int32)
    xt = xf.T

    hv_sc = _sc_encode(level_weight, position_weight, idx)

    hv_tc = pl.pallas_call(
        _tc_encode_body,
        grid=(DTC // DBLK,),
        in_specs=[
            pl.BlockSpec((POS, BATCH), lambda i: (0, 0)),
            pl.BlockSpec((LEVELS, DBLK), lambda i: (0, i)),
            pl.BlockSpec((POS, DBLK), lambda i: (0, i)),
        ],
        out_specs=pl.BlockSpec((BATCH, DBLK), lambda i: (0, i)),
        out_shape=jax.ShapeDtypeStruct((BATCH, DTC), jnp.float32),
    )(xt, level_weight, position_weight)

    logit = pl.pallas_call(
        _finish_body,
        out_shape=jax.ShapeDtypeStruct((BATCH, CLASSES), jnp.float32),
    )(hv_tc, hv_sc, classify_weight)
    return logit


# speedup vs baseline: 4.5061x; 1.0248x over previous
import functools

import jax
import jax.numpy as jnp
from jax import lax
from jax.experimental import pallas as pl
from jax.experimental.pallas import tpu as pltpu
from jax.experimental.pallas import tpu_sc as plsc

DIMS = 10000
LEVELS = 1000
POS = 784
BATCH = 8
CLASSES = 10

W = 80
NCHUNK = DIMS // W
NTILES = 32
PSTRIP = 56
NSTRIP = POS // PSTRIP
NV = W // 16

DTC = 8320
DBLK = 640
C0 = DTC // W
NSC = NCHUNK - C0


def _sc_encode_body(lev_hbm, pos_hbm, idx_hbm, out_hbm,
                    idxb, ilist, lrA, lrB, ptA, ptB, accb, semA, semB):
    wid = lax.axis_index("s") * 2 + lax.axis_index("c")
    nfull = NSC // NTILES
    nrem = NSC % NTILES
    nchunks = nfull + jnp.where(wid < nrem, 1, 0)
    pltpu.sync_copy(idx_hbm, idxb)

    def chunk_body(ci, carry0):
        c = C0 + wid + NTILES * ci

        for b in range(BATCH):
            for v in range(POS // 16):
                sl = pl.ds(v * 16, 16)
                ilist[b, sl] = idxb[b, sl] * NCHUNK + c

        zero = jnp.zeros((16,), jnp.float32)
        for b in range(BATCH):
            for j in range(NV):
                accb[b, pl.ds(j * 16, 16)] = zero

        def copies(s, lr, pt, sem):
            cps = [
                pltpu.make_async_copy(
                    lev_hbm.at[ilist.at[b, pl.ds(s * PSTRIP, PSTRIP)]],
                    lr.at[b], sem)
                for b in range(BATCH)
            ]
            cps.append(pltpu.make_async_copy(
                pos_hbm.at[pl.ds(s * PSTRIP, PSTRIP), pl.ds(c * W, W)],
                pt, sem))
            return cps

        def issue(s, lr, pt, sem):
            for cp in copies(s, lr, pt, sem):
                cp.start()

        def drain(s, lr, pt, sem):
            for cp in copies(s, lr, pt, sem):
                cp.wait()

        def compute(lr, pt):
            def pbody(p, carry2):
                pv = [pt[p, pl.ds(j * 16, 16)] for j in range(NV)]
                for b in range(BATCH):
                    for j in range(NV):
                        lv = lr[b, p, pl.ds(j * 16, 16)]
                        plsc.addupdate(accb.at[b, pl.ds(j * 16, 16)],
                                       lv * pv[j])
                return carry2

            lax.fori_loop(0, PSTRIP, pbody, 0)

        issue(0, lrA, ptA, semA)

        def ring(g, carry1):
            s0 = 2 * g
            issue(s0 + 1, lrB, ptB, semB)
            drain(s0, lrA, ptA, semA)
            compute(lrA, ptA)

            @pl.when(s0 + 2 < NSTRIP)
            def _():
                issue(s0 + 2, lrA, ptA, semA)

            drain(s0 + 1, lrB, ptB, semB)
            compute(lrB, ptB)
            return carry1

        lax.fori_loop(0, NSTRIP // 2, ring, 0)
        pltpu.sync_copy(accb, out_hbm.at[:, pl.ds((c - C0) * W, W)])
        return carry0

    lax.fori_loop(0, nchunks, chunk_body, 0)


def _sc_encode(level_weight, position_weight, idx):
    lev_rows = level_weight.reshape(LEVELS * NCHUNK, W)
    mesh = plsc.VectorSubcoreMesh(core_axis_name="c", subcore_axis_name="s")
    sc_encode = pl.kernel(
        _sc_encode_body,
        out_type=jax.ShapeDtypeStruct((BATCH, DIMS - DTC), jnp.float32),
        mesh=mesh,
        compiler_params=pltpu.CompilerParams(use_tc_tiling_on_sc=False),
        scratch_types=[
            pltpu.VMEM((BATCH, POS), jnp.int32),
            pltpu.VMEM((BATCH, POS), jnp.int32),
            pltpu.VMEM((BATCH, PSTRIP, W), jnp.float32),
            pltpu.VMEM((BATCH, PSTRIP, W), jnp.float32),
            pltpu.VMEM((PSTRIP, W), jnp.float32),
            pltpu.VMEM((PSTRIP, W), jnp.float32),
            pltpu.VMEM((BATCH, W), jnp.float32),
            pltpu.SemaphoreType.DMA,
            pltpu.SemaphoreType.DMA,
        ],
    )
    return sc_encode(lev_rows, position_weight, idx)


sc_encode_for_test = jax.jit(_sc_encode)


def _tc_encode_body(xt_ref, lev_ref, pos_ref, out_ref):
    lev_f8 = lev_ref[...].astype(jnp.float8_e4m3fn)
    pos_blk = pos_ref[...]
    lvl_iota = jax.lax.broadcasted_iota(jnp.int32, (POS, LEVELS), 1)
    for b in range(BATCH):
        xb = xt_ref[:, b : b + 1]
        idx = jnp.clip(jnp.round(xb * (LEVELS - 1)), 0, LEVELS - 1).astype(jnp.int32)
        onehot = (idx == lvl_iota).astype(jnp.float8_e4m3fn)
        g = jnp.dot(onehot, lev_f8, preferred_element_type=jnp.float32)
        out_ref[b, :] = jnp.sum(g * pos_blk, axis=0)


def _finish_body(hv_tc_ref, hv_sc_ref, cw_ref, out_ref):
    cw = cw_ref[...]
    enc_tc = jnp.where(hv_tc_ref[...] > 0, 1.0, -1.0).astype(jnp.float32)
    enc_sc = jnp.where(hv_sc_ref[...] > 0, 1.0, -1.0).astype(jnp.float32)
    out_ref[...] = (
        jax.lax.dot_general(enc_tc, cw[:, :DTC], (((1,), (1,)), ((), ())),
                            preferred_element_type=jnp.float32)
        + jax.lax.dot_general(enc_sc, cw[:, DTC:], (((1,), (1,)), ((), ())),
                              preferred_element_type=jnp.float32))


@jax.jit
def kernel(x, position_weight, level_weight, classify_weight):
    xf = x.reshape(BATCH, POS)
    idx = jnp.clip(jnp.round(xf * (LEVELS - 1)), 0, LEVELS - 1).astype(jnp.int32)
    xt = xf.T

    hv_sc = _sc_encode(level_weight, position_weight, idx)

    hv_tc = pl.pallas_call(
        _tc_encode_body,
        grid=(DTC // DBLK,),
        in_specs=[
            pl.BlockSpec((POS, BATCH), lambda i: (0, 0)),
            pl.BlockSpec((LEVELS, DBLK), lambda i: (0, i)),
            pl.BlockSpec((POS, DBLK), lambda i: (0, i)),
        ],
        out_specs=pl.BlockSpec((BATCH, DBLK), lambda i: (0, i)),
        out_shape=jax.ShapeDtypeStruct((BATCH, DTC), jnp.float32),
    )(xt, level_weight, position_weight)

    logit = pl.pallas_call(
        _finish_body,
        out_shape=jax.ShapeDtypeStruct((BATCH, CLASSES), jnp.float32),
    )(hv_tc, hv_sc, classify_weight)
    return logit


# speedup vs baseline: 11.5405x; 2.5611x over previous
import functools

import jax
import jax.numpy as jnp
from jax import lax
from jax.experimental import pallas as pl
from jax.experimental.pallas import tpu as pltpu
from jax.experimental.pallas import tpu_sc as plsc

DIMS = 10000
LEVELS = 1000
POS = 784
BATCH = 8
CLASSES = 10

W = 80
NCHUNK = DIMS // W
NTILES = 32
PSTRIP = 56
NSTRIP = POS // PSTRIP
NV = W // 16

DTC = 10000
DBLK = 640
C0 = DTC // W
NSC = NCHUNK - C0


def _sc_encode_body(lev_hbm, pos_hbm, idx_hbm, out_hbm,
                    idxb, ilist, lrA, lrB, ptA, ptB, accb, semA, semB):
    wid = lax.axis_index("s") * 2 + lax.axis_index("c")
    nfull = NSC // NTILES
    nrem = NSC % NTILES
    nchunks = nfull + jnp.where(wid < nrem, 1, 0)
    pltpu.sync_copy(idx_hbm, idxb)

    def chunk_body(ci, carry0):
        c = C0 + wid + NTILES * ci

        for b in range(BATCH):
            for v in range(POS // 16):
                sl = pl.ds(v * 16, 16)
                ilist[b, sl] = idxb[b, sl] * NCHUNK + c

        zero = jnp.zeros((16,), jnp.float32)
        for b in range(BATCH):
            for j in range(NV):
                accb[b, pl.ds(j * 16, 16)] = zero

        def copies(s, lr, pt, sem):
            cps = [
                pltpu.make_async_copy(
                    lev_hbm.at[ilist.at[b, pl.ds(s * PSTRIP, PSTRIP)]],
                    lr.at[b], sem)
                for b in range(BATCH)
            ]
            cps.append(pltpu.make_async_copy(
                pos_hbm.at[pl.ds(s * PSTRIP, PSTRIP), pl.ds(c * W, W)],
                pt, sem))
            return cps

        def issue(s, lr, pt, sem):
            for cp in copies(s, lr, pt, sem):
                cp.start()

        def drain(s, lr, pt, sem):
            for cp in copies(s, lr, pt, sem):
                cp.wait()

        def compute(lr, pt):
            def pbody(p, carry2):
                pv = [pt[p, pl.ds(j * 16, 16)] for j in range(NV)]
                for b in range(BATCH):
                    for j in range(NV):
                        lv = lr[b, p, pl.ds(j * 16, 16)]
                        plsc.addupdate(accb.at[b, pl.ds(j * 16, 16)],
                                       lv * pv[j])
                return carry2

            lax.fori_loop(0, PSTRIP, pbody, 0)

        issue(0, lrA, ptA, semA)

        def ring(g, carry1):
            s0 = 2 * g
            issue(s0 + 1, lrB, ptB, semB)
            drain(s0, lrA, ptA, semA)
            compute(lrA, ptA)

            @pl.when(s0 + 2 < NSTRIP)
            def _():
                issue(s0 + 2, lrA, ptA, semA)

            drain(s0 + 1, lrB, ptB, semB)
            compute(lrB, ptB)
            return carry1

        lax.fori_loop(0, NSTRIP // 2, ring, 0)
        pltpu.sync_copy(accb, out_hbm.at[:, pl.ds((c - C0) * W, W)])
        return carry0

    lax.fori_loop(0, nchunks, chunk_body, 0)


def _sc_encode(level_weight, position_weight, idx):
    lev_rows = level_weight.reshape(LEVELS * NCHUNK, W)
    mesh = plsc.VectorSubcoreMesh(core_axis_name="c", subcore_axis_name="s")
    sc_encode = pl.kernel(
        _sc_encode_body,
        out_type=jax.ShapeDtypeStruct((BATCH, DIMS - DTC), jnp.float32),
        mesh=mesh,
        compiler_params=pltpu.CompilerParams(use_tc_tiling_on_sc=False),
        scratch_types=[
            pltpu.VMEM((BATCH, POS), jnp.int32),
            pltpu.VMEM((BATCH, POS), jnp.int32),
            pltpu.VMEM((BATCH, PSTRIP, W), jnp.float32),
            pltpu.VMEM((BATCH, PSTRIP, W), jnp.float32),
            pltpu.VMEM((PSTRIP, W), jnp.float32),
            pltpu.VMEM((PSTRIP, W), jnp.float32),
            pltpu.VMEM((BATCH, W), jnp.float32),
            pltpu.SemaphoreType.DMA,
            pltpu.SemaphoreType.DMA,
        ],
    )
    return sc_encode(lev_rows, position_weight, idx)


sc_encode_for_test = jax.jit(_sc_encode)


def _tc_encode_body(xt_ref, lev_ref, pos_ref, out_ref):
    lev_f8 = lev_ref[...].astype(jnp.float8_e4m3fn)
    pos_blk = pos_ref[...]
    lvl_iota = jax.lax.broadcasted_iota(jnp.int32, (POS, LEVELS), 1)
    for b in range(BATCH):
        xb = xt_ref[:, b : b + 1]
        idx = jnp.clip(jnp.round(xb * (LEVELS - 1)), 0, LEVELS - 1).astype(jnp.int32)
        onehot = (idx == lvl_iota).astype(jnp.float8_e4m3fn)
        g = jnp.dot(onehot, lev_f8, preferred_element_type=jnp.float32)
        out_ref[b, :] = jnp.sum(g * pos_blk, axis=0)


def _finish_tc_only_body(hv_tc_ref, cw_ref, out_ref):
    enc = jnp.where(hv_tc_ref[...] > 0, 1.0, -1.0).astype(jnp.float32)
    out_ref[...] = jax.lax.dot_general(
        enc, cw_ref[...], (((1,), (1,)), ((), ())),
        preferred_element_type=jnp.float32)


def _finish_body(hv_tc_ref, hv_sc_ref, cw_ref, out_ref):
    cw = cw_ref[...]
    enc_tc = jnp.where(hv_tc_ref[...] > 0, 1.0, -1.0).astype(jnp.float32)
    enc_sc = jnp.where(hv_sc_ref[...] > 0, 1.0, -1.0).astype(jnp.float32)
    out_ref[...] = (
        jax.lax.dot_general(enc_tc, cw[:, :DTC], (((1,), (1,)), ((), ())),
                            preferred_element_type=jnp.float32)
        + jax.lax.dot_general(enc_sc, cw[:, DTC:], (((1,), (1,)), ((), ())),
                              preferred_element_type=jnp.float32))


@jax.jit
def kernel(x, position_weight, level_weight, classify_weight):
    xf = x.reshape(BATCH, POS)
    idx = jnp.clip(jnp.round(xf * (LEVELS - 1)), 0, LEVELS - 1).astype(jnp.int32)
    xt = xf.T

    if NSC:
        hv_sc = _sc_encode(level_weight, position_weight, idx)
    hv_tc = pl.pallas_call(
        _tc_encode_body,
        grid=(DTC // DBLK,),
        in_specs=[
            pl.BlockSpec((POS, BATCH), lambda i: (0, 0)),
            pl.BlockSpec((LEVELS, DBLK), lambda i: (0, i)),
            pl.BlockSpec((POS, DBLK), lambda i: (0, i)),
        ],
        out_specs=pl.BlockSpec((BATCH, DBLK), lambda i: (0, i)),
        out_shape=jax.ShapeDtypeStruct((BATCH, DTC), jnp.float32),
    )(xt, level_weight, position_weight)

    if NSC:
        logit = pl.pallas_call(
            _finish_body,
            out_shape=jax.ShapeDtypeStruct((BATCH, CLASSES), jnp.float32),
        )(hv_tc, hv_sc, classify_weight)
    else:
        logit = pl.pallas_call(
            _finish_tc_only_body,
            out_shape=jax.ShapeDtypeStruct((BATCH, CLASSES), jnp.float32),
        )(hv_tc, classify_weight)
    return logit
